# baseline (speedup 1.0000x reference)
import functools

import jax
import jax.numpy as jnp
from jax import lax
from jax.experimental import pallas as pl
from jax.experimental.pallas import tpu as pltpu
from jax.experimental.pallas import tpu_sc as plsc

_NC = 2
_NS = 16
_NW = _NC * _NS
_L = 16
_GATHER_ROWS = 128


def _build_gather(F, V, D, R):
    rpw = R // _NW
    chunk = 512
    n_chunk = rpw // chunk
    gpc = chunk // _GATHER_ROWS
    n_off = 13
    slices_per_worker = rpw // _L
    n_outer = slices_per_worker // n_off

    mesh = plsc.VectorSubcoreMesh(core_axis_name="c", subcore_axis_name="s")

    @functools.partial(
        pl.kernel,
        mesh=mesh,
        compiler_params=pltpu.CompilerParams(use_tc_tiling_on_sc=False),
        out_type=jax.ShapeDtypeStruct((R, D), jnp.float32),
        scratch_types=[
            pltpu.VMEM((rpw,), jnp.int32),
            pltpu.VMEM((chunk, D), jnp.float32),
            pltpu.SemaphoreType.DMA,
        ],
    )
    def gather_kernel(tab_hbm, idx_hbm, out_hbm, idx_v, rows_v, sem):
        wid = lax.axis_index("s") * _NC + lax.axis_index("c")
        base = wid * rpw

        pltpu.sync_copy(idx_hbm.at[pl.ds(base, rpw)], idx_v)

        iota = lax.iota(jnp.int32, _L)
        offs = [((iota + _L * l) % F) * V for l in range(n_off)]

        def offset_body(o, carry):
            for l in range(n_off):
                s = o * n_off + l
                sl = pl.ds(s * _L, _L)
                idx_v[sl] = idx_v[sl] + offs[l]
            return carry

        lax.fori_loop(0, n_outer, offset_body, 0, unroll=False)

        def chunk_body(c, carry):
            rb = c * chunk
            copies = [
                pltpu.async_copy(
                    tab_hbm.at[idx_v.at[pl.ds(rb + g * _GATHER_ROWS, _GATHER_ROWS)]],
                    rows_v.at[pl.ds(g * _GATHER_ROWS, _GATHER_ROWS)],
                    sem,
                )
                for g in range(gpc)
            ]
            for cp in copies:
                cp.wait()
            pltpu.sync_copy(rows_v, out_hbm.at[pl.ds(base + rb, chunk)])
            return carry

        lax.fori_loop(0, n_chunk, chunk_body, 0, unroll=False)

    return gather_kernel


def kernel(inputs, tables):
    F, V, D = tables.shape
    B = inputs.shape[0]
    R = B * F
    tab_flat = tables.reshape(F * V, D)
    idx_flat = inputs.reshape(R).astype(jnp.int32)
    out = _build_gather(F, V, D, R)(tab_flat, idx_flat)
    return out.reshape(B, F, D)


# speedup vs baseline: 1.0115x; 1.0115x over previous
import functools

import jax
import jax.numpy as jnp
from jax import lax
from jax.experimental import pallas as pl
from jax.experimental.pallas import tpu as pltpu
from jax.experimental.pallas import tpu_sc as plsc

_NC = 2
_NS = 16
_NW = _NC * _NS
_L = 16
_GATHER_ROWS = 128


def _build_gather(F, V, D, R):
    rpw = R // _NW
    chunk = 1664
    n_chunk = rpw // chunk
    gpc = chunk // _GATHER_ROWS
    n_off = 13
    slices_per_worker = rpw // _L
    n_outer = slices_per_worker // n_off

    mesh = plsc.VectorSubcoreMesh(core_axis_name="c", subcore_axis_name="s")

    @functools.partial(
        pl.kernel,
        mesh=mesh,
        compiler_params=pltpu.CompilerParams(use_tc_tiling_on_sc=False),
        out_type=jax.ShapeDtypeStruct((R, D), jnp.float32),
        scratch_types=[
            pltpu.VMEM((rpw,), jnp.int32),
            pltpu.VMEM((chunk, D), jnp.float32),
            pltpu.VMEM((chunk, D), jnp.float32),
            pltpu.SemaphoreType.DMA,
            pltpu.SemaphoreType.DMA,
            pltpu.SemaphoreType.DMA,
            pltpu.SemaphoreType.DMA,
        ],
    )
    def gather_kernel(tab_hbm, idx_hbm, out_hbm, idx_v, buf_a, buf_b,
                      gsem_a, gsem_b, osem_a, osem_b):
        wid = lax.axis_index("s") * _NC + lax.axis_index("c")
        base = wid * rpw

        pltpu.sync_copy(idx_hbm.at[pl.ds(base, rpw)], idx_v)

        iota = lax.iota(jnp.int32, _L)
        offs = [((iota + _L * l) % F) * V for l in range(n_off)]

        def offset_body(o, carry):
            for l in range(n_off):
                s = o * n_off + l
                sl = pl.ds(s * _L, _L)
                idx_v[sl] = idx_v[sl] + offs[l]
            return carry

        lax.fori_loop(0, n_outer, offset_body, 0, unroll=False)

        bufs = (buf_a, buf_b)
        gsems = (gsem_a, gsem_b)
        osems = (osem_a, osem_b)
        gathers = [None] * n_chunk
        outs = [None] * n_chunk

        def fire(c):
            rb = c * chunk
            buf, gsem = bufs[c % 2], gsems[c % 2]
            gathers[c] = [
                pltpu.async_copy(
                    tab_hbm.at[idx_v.at[pl.ds(rb + g * _GATHER_ROWS, _GATHER_ROWS)]],
                    buf.at[pl.ds(g * _GATHER_ROWS, _GATHER_ROWS)],
                    gsem,
                )
                for g in range(gpc)
            ]

        def writeback(c):
            for h in gathers[c]:
                h.wait()
            outs[c] = pltpu.async_copy(
                bufs[c % 2], out_hbm.at[pl.ds(base + c * chunk, chunk)],
                osems[c % 2])

        for c in range(n_chunk):
            if c >= 2:
                outs[c - 2].wait()
            fire(c)
            if c >= 1:
                writeback(c - 1)
        writeback(n_chunk - 1)
        outs[n_chunk - 2].wait()
        outs[n_chunk - 1].wait()

    return gather_kernel


def kernel(inputs, tables):
    F, V, D = tables.shape
    B = inputs.shape[0]
    R = B * F
    tab_flat = tables.reshape(F * V, D)
    idx_flat = inputs.reshape(R).astype(jnp.int32)
    out = _build_gather(F, V, D, R)(tab_flat, idx_flat)
    return out.reshape(B, F, D)


# speedup vs baseline: 3.8067x; 3.7633x over previous
import functools

import jax
import jax.numpy as jnp
from jax import lax
from jax.experimental import pallas as pl
from jax.experimental.pallas import tpu as pltpu
from jax.experimental.pallas import tpu_sc as plsc

_NC = 2
_NS = 16
_NW = _NC * _NS
_L = 16


def _build_plane_gather(F, V, D, B):
    n_planes = F * D
    ppw = n_planes // _NW
    hb = B // 2

    mesh = plsc.VectorSubcoreMesh(core_axis_name="c", subcore_axis_name="s")

    @functools.partial(
        pl.kernel,
        mesh=mesh,
        compiler_params=pltpu.CompilerParams(needs_layout_passes=False),
        out_type=jax.ShapeDtypeStruct((n_planes, B), jnp.float32),
        scratch_types=[
            pltpu.VMEM((V,), jnp.float32),
            pltpu.VMEM((hb,), jnp.int32),
            pltpu.VMEM((hb,), jnp.float32),
        ],
    )
    def plane_kernel(tab_hbm, idx_hbm, out_hbm, plane_v, idx_v, row_v):
        wid = lax.axis_index("s") * _NC + lax.axis_index("c")
        r0 = wid * ppw

        def body(i, carry):
            r = r0 + i
            f = r // D
            d = r % D
            pltpu.sync_copy(tab_hbm.at[f, d, :], plane_v)

            def half(h, c):
                pltpu.sync_copy(idx_hbm.at[pl.ds(f * B + h * hb, hb)], idx_v)

                def gloop(j, c2):
                    sl = pl.ds(j * _L, _L)
                    row_v[sl] = plsc.load_gather(plane_v, [idx_v[sl]])
                    return c2

                lax.fori_loop(0, hb // _L, gloop, 0, unroll=False)
                pltpu.sync_copy(row_v, out_hbm.at[r, pl.ds(h * hb, hb)])
                return c

            lax.fori_loop(0, 2, half, 0, unroll=False)
            return carry

        lax.fori_loop(0, ppw, body, 0, unroll=False)

    return plane_kernel


def kernel(inputs, tables):
    F, V, D = tables.shape
    B = inputs.shape[0]
    tab_t = jnp.transpose(tables, (0, 2, 1))
    idx_f = jnp.transpose(inputs, (1, 0)).reshape(F * B)
    out = _build_plane_gather(F, V, D, B)(tab_t, idx_f.astype(jnp.int32))
    return out.reshape(F, D, B).transpose(2, 0, 1)
